# baseline (speedup 1.0000x reference)
import functools

import jax
import jax.numpy as jnp
from jax import lax
from jax.experimental import pallas as pl
from jax.experimental.pallas import tpu as pltpu
from jax.experimental.pallas import tpu_sc as plsc

B = 16384
ED = 64
RD = 32
NC = 2
NS = 16
L = 16
NW = NC * NS
BPW = B // NW
GROUPS = BPW // L


def _body(head_r, rel_r, tail_r, ent_r, remb_r, tmat_r, out_r,
          hidx, ridx, tidx, h_v, t_v, r_v, m_v, score_v, sem):
    c = lax.axis_index("c")
    s = lax.axis_index("s")
    wid = s * NC + c
    row0 = wid * GROUPS

    pltpu.sync_copy(head_r.at[pl.ds(row0, GROUPS)], hidx)
    pltpu.sync_copy(rel_r.at[pl.ds(row0, GROUPS)], ridx)
    pltpu.sync_copy(tail_r.at[pl.ds(row0, GROUPS)], tidx)

    lane = lax.iota(jnp.int32, L)

    def group(g, carry):
        hi = hidx[g]
        ti = tidx[g]
        ri = ridx[g]
        ch = pltpu.async_copy(ent_r.at[hi], h_v, sem)
        ct = pltpu.async_copy(ent_r.at[ti], t_v, sem)
        cr = pltpu.async_copy(remb_r.at[ri], r_v, sem)
        cm = pltpu.async_copy(tmat_r.at[ri], m_v, sem)
        ch.wait()
        ct.wait()
        cr.wait()
        cm.wait()

        def dstep(d, accs):
            dd = jnp.zeros((L,), jnp.int32) + d
            hT = plsc.load_gather(h_v, [lane, dd])
            tT = plsc.load_gather(t_v, [lane, dd])
            diff = hT - tT
            out = []
            for j in range(RD):
                m = plsc.load_gather(m_v, [lane, dd + (j * ED)])
                out.append(accs[j] + m * diff)
            return tuple(out)

        accs = lax.fori_loop(
            0, ED, dstep,
            tuple(jnp.zeros((L,), jnp.float32) for _ in range(RD)))

        nrm = jnp.zeros((L,), jnp.float32)
        for j in range(RD):
            rT = plsc.load_gather(r_v, [lane, jnp.full((L,), j, jnp.int32)])
            sc = accs[j] + rT
            nrm = nrm + sc * sc

        x = jnp.maximum(nrm, jnp.float32(1e-30))
        i = plsc.bitcast(x, jnp.int32)
        i = 0x5F3759DF - lax.shift_right_logical(i, 1)
        y = plsc.bitcast(i, jnp.float32)
        for _ in range(3):
            y = y * (jnp.float32(1.5) - jnp.float32(0.5) * x * y * y)
        score_v[g] = -(x * y)
        return carry

    lax.fori_loop(0, GROUPS, group, 0)
    pltpu.sync_copy(score_v, out_r.at[pl.ds(row0, GROUPS)])


@jax.jit
def _transr_sc(head2, rel2, tail2, entity_emb, relation_emb, transfer_mat):
    mesh = plsc.VectorSubcoreMesh(
        core_axis_name="c", subcore_axis_name="s",
        num_cores=NC, num_subcores=NS)
    fn = pl.kernel(
        _body,
        out_type=jax.ShapeDtypeStruct((B // L, L), jnp.float32),
        mesh=mesh,
        compiler_params=pltpu.CompilerParams(
            needs_layout_passes=False, use_tc_tiling_on_sc=False),
        scratch_types=[
            pltpu.VMEM((GROUPS, L), jnp.int32),
            pltpu.VMEM((GROUPS, L), jnp.int32),
            pltpu.VMEM((GROUPS, L), jnp.int32),
            pltpu.VMEM((L, ED), jnp.float32),
            pltpu.VMEM((L, ED), jnp.float32),
            pltpu.VMEM((L, RD), jnp.float32),
            pltpu.VMEM((L, RD * ED), jnp.float32),
            pltpu.VMEM((GROUPS, L), jnp.float32),
            pltpu.SemaphoreType.DMA,
        ],
    )
    return fn(head2, rel2, tail2, entity_emb, relation_emb, transfer_mat)


def kernel(head, relation, tail, entity_emb, relation_emb, transfer_mat):
    out2 = _transr_sc(
        head.reshape(B // L, L), relation.reshape(B // L, L),
        tail.reshape(B // L, L), entity_emb, relation_emb, transfer_mat)
    return out2.reshape(B)


# speedup vs baseline: 1.1947x; 1.1947x over previous
import jax
import jax.numpy as jnp
from jax import lax
from jax.experimental import pallas as pl
from jax.experimental.pallas import tpu as pltpu
from jax.experimental.pallas import tpu_sc as plsc

B = 16384
ED = 64
RD = 32
NC = 2
NS = 16
L = 16
NW = NC * NS
BPW = B // NW
GROUPS = BPW // L
IDXROWS = BPW // 128


def _issue(g, refs, bufs, midx, lane, sem):
    head_r, rel_r, tail_r, ent_r, remb_r, tmat_r = refs
    hidx, ridx, tidx, h_v, t_v, r_v, m_v = bufs
    p = g * L + lane
    prow = lax.shift_right_logical(p, 7)
    pcol = lax.bitwise_and(p, 127)
    hi = plsc.load_gather(hidx, [prow, pcol])
    ti = plsc.load_gather(tidx, [prow, pcol])
    ri = plsc.load_gather(ridx, [prow, pcol])
    copies = [
        pltpu.async_copy(ent_r.at[lax.shift_right_logical(hi, 1)], h_v, sem),
        pltpu.async_copy(ent_r.at[lax.shift_right_logical(ti, 1)], t_v, sem),
        pltpu.async_copy(remb_r.at[lax.shift_right_logical(ri, 2)], r_v, sem),
        pltpu.async_copy(tmat_r.at[midx.at[g, 0]], m_v.at[pl.ds(0, 8 * L)], sem),
        pltpu.async_copy(tmat_r.at[midx.at[g, 1]], m_v.at[pl.ds(8 * L, 8 * L)], sem),
    ]
    return copies


def _wait(refs, bufs, sem):
    head_r, rel_r, tail_r, ent_r, remb_r, tmat_r = refs
    hidx, ridx, tidx, h_v, t_v, r_v, m_v = bufs
    pltpu.make_async_copy(ent_r.at[pl.ds(0, L)], h_v, sem).wait()
    pltpu.make_async_copy(ent_r.at[pl.ds(0, L)], t_v, sem).wait()
    pltpu.make_async_copy(remb_r.at[pl.ds(0, L)], r_v, sem).wait()
    pltpu.make_async_copy(
        tmat_r.at[pl.ds(0, 8 * L)], m_v.at[pl.ds(0, 8 * L)], sem).wait()
    pltpu.make_async_copy(
        tmat_r.at[pl.ds(0, 8 * L)], m_v.at[pl.ds(8 * L, 8 * L)], sem).wait()


def _compute(g, refs, bufs, lane, lane16, dT, score_v):
    hidx, ridx, tidx, h_v, t_v, r_v, m_v = bufs
    p = g * L + lane
    prow = lax.shift_right_logical(p, 7)
    pcol = lax.bitwise_and(p, 127)
    hi = plsc.load_gather(hidx, [prow, pcol])
    ti = plsc.load_gather(tidx, [prow, pcol])
    ri = plsc.load_gather(ridx, [prow, pcol])
    hcol = lax.bitwise_and(hi, 1) * ED
    tcol = lax.bitwise_and(ti, 1) * ED
    rcol = lax.bitwise_and(ri, 3) * RD

    def dpre(d, carry):
        hT = plsc.load_gather(h_v, [lane, hcol + d])
        tT = plsc.load_gather(t_v, [lane, tcol + d])
        dT[d] = hT - tT
        return carry

    lax.fori_loop(0, ED, dpre, 0)

    nrm = jnp.zeros((L,), jnp.float32)
    for jb in range(4):
        j0 = jb * 8
        rows = [lane16 + ((j0 + jj) >> 1) for jj in range(8)]

        def dstep(d, accs, rows=rows):
            dvec = dT[d]
            c0 = jnp.zeros((L,), jnp.int32) + d
            c1 = c0 + ED
            out = []
            for jj in range(8):
                col = c1 if ((j0 + jj) & 1) else c0
                m = plsc.load_gather(m_v, [rows[jj], col])
                out.append(accs[jj] + m * dvec)
            return tuple(out)

        accs = lax.fori_loop(
            0, ED, dstep,
            tuple(jnp.zeros((L,), jnp.float32) for _ in range(8)))
        for jj in range(8):
            rT = plsc.load_gather(r_v, [lane, rcol + (j0 + jj)])
            sc = accs[jj] + rT
            nrm = nrm + sc * sc

    x = jnp.maximum(nrm, jnp.float32(1e-30))
    i = plsc.bitcast(x, jnp.int32)
    i = 0x5F3759DF - lax.shift_right_logical(i, 1)
    y = plsc.bitcast(i, jnp.float32)
    for _ in range(3):
        y = y * (jnp.float32(1.5) - jnp.float32(0.5) * x * y * y)
    res = -(x * y)
    srow = lax.shift_right_logical(g * L, 7)
    scol = lax.bitwise_and(g * L, 127)
    plsc.store_scatter(score_v, [jnp.full((L,), srow, jnp.int32),
                                 scol + lane], res)


def _body(head_r, rel_r, tail_r, ent_r, remb_r, tmat_r, out_r,
          hidx, ridx, tidx,
          h0, t0, r0, m0,
          h1, t1, r1, m1,
          midx, dT, score_v, sem0, sem1):
    c = lax.axis_index("c")
    s = lax.axis_index("s")
    wid = s * NC + c
    row0 = wid * IDXROWS

    pltpu.sync_copy(head_r.at[pl.ds(row0, IDXROWS)], hidx)
    pltpu.sync_copy(rel_r.at[pl.ds(row0, IDXROWS)], ridx)
    pltpu.sync_copy(tail_r.at[pl.ds(row0, IDXROWS)], tidx)

    lane = lax.iota(jnp.int32, L)
    lane16 = lane * L
    refs = (head_r, rel_r, tail_r, ent_r, remb_r, tmat_r)
    bufs0 = (hidx, ridx, tidx, h0, t0, r0, m0)
    bufs1 = (hidx, ridx, tidx, h1, t1, r1, m1)

    def buildm(g, carry):
        for s_ in range(L):
            ps = g * L + s_
            rs = plsc.load_gather(
                ridx,
                [jnp.full((L,), lax.shift_right_logical(ps, 7), jnp.int32),
                 jnp.full((L,), lax.bitwise_and(ps, 127), jnp.int32)])
            vals = rs * L + lane
            midx[g, s_ // 8, pl.ds((s_ % 8) * L, L)] = vals
        return carry

    lax.fori_loop(0, GROUPS, buildm, 0)

    _issue(0, refs, bufs0, midx, lane, sem0)

    def step(gg, carry):
        g0 = gg * 2
        _issue(g0 + 1, refs, bufs1, midx, lane, sem1)
        _wait(refs, bufs0, sem0)
        _compute(g0, refs, bufs0, lane, lane16, dT, score_v)

        @pl.when(gg < GROUPS // 2 - 1)
        def _():
            _issue(g0 + 2, refs, bufs0, midx, lane, sem0)

        _wait(refs, bufs1, sem1)
        _compute(g0 + 1, refs, bufs1, lane, lane16, dT, score_v)
        return carry

    lax.fori_loop(0, GROUPS // 2, step, 0)
    pltpu.sync_copy(score_v, out_r.at[pl.ds(row0, IDXROWS)])


@jax.jit
def _transr_sc(head2, rel2, tail2, ent2, remb2, tmat2):
    mesh = plsc.VectorSubcoreMesh(
        core_axis_name="c", subcore_axis_name="s",
        num_cores=NC, num_subcores=NS)
    dbl = lambda: [
        pltpu.VMEM((L, 128), jnp.float32),
        pltpu.VMEM((L, 128), jnp.float32),
        pltpu.VMEM((L, 128), jnp.float32),
        pltpu.VMEM((16 * L, 128), jnp.float32),
    ]
    fn = pl.kernel(
        _body,
        out_type=jax.ShapeDtypeStruct((128, 128), jnp.float32),
        mesh=mesh,
        compiler_params=pltpu.CompilerParams(
            needs_layout_passes=False, use_tc_tiling_on_sc=False),
        scratch_types=[
            pltpu.VMEM((IDXROWS, 128), jnp.int32),
            pltpu.VMEM((IDXROWS, 128), jnp.int32),
            pltpu.VMEM((IDXROWS, 128), jnp.int32),
            *dbl(), *dbl(),
            pltpu.VMEM((GROUPS, 2, 128), jnp.int32),
            pltpu.VMEM((ED, L), jnp.float32),
            pltpu.VMEM((IDXROWS, 128), jnp.float32),
            pltpu.SemaphoreType.DMA,
            pltpu.SemaphoreType.DMA,
        ],
    )
    return fn(head2, rel2, tail2, ent2, remb2, tmat2)


def kernel(head, relation, tail, entity_emb, relation_emb, transfer_mat):
    out2 = _transr_sc(
        head.reshape(128, 128), relation.reshape(128, 128),
        tail.reshape(128, 128),
        entity_emb.reshape(500000, 128),
        relation_emb.reshape(250, 128),
        transfer_mat.reshape(16000, 128))
    return out2.reshape(B)
